# baseline (speedup 1.0000x reference)
import jax
import jax.numpy as jnp
from jax import lax
from jax.experimental import pallas as pl
from jax.experimental.pallas import tpu as pltpu
from jax.experimental.pallas import tpu_sc as plsc

NC = 2
NS = 16
LN = 16

B = 112
NB = 90
EPT = NB * B
EPAD = NC * NS * EPT

NPAD = 10240
NPT = NPAD // NS


def _sc_deg_body(col_hbm, ew_hbm, deg_out, col_v, ew_v, zvec, deg_sh,
                 sem_d):
    c = lax.axis_index("c")
    s = lax.axis_index("s")

    pltpu.sync_copy(col_hbm.at[s, c], col_v)
    pltpu.sync_copy(ew_hbm.at[s, c], ew_v)

    def zero_zvec(i, _):
        zvec[pl.ds(i * LN, LN)] = jnp.zeros((LN,), jnp.float32)
        return 0
    lax.fori_loop(0, NPT // LN, zero_zvec, 0)
    pltpu.sync_copy(zvec, deg_sh.at[pl.ds(s * NPT, NPT)])
    plsc.subcore_barrier()

    def deg_batch(g, _):
        pltpu.async_copy(ew_v.at[g], deg_sh.at[col_v.at[g]], sem_d, add=True)
        return 0
    lax.fori_loop(0, NB, deg_batch, 0)
    def deg_drain(g, _):
        pltpu.make_async_copy(ew_v.at[g], deg_sh.at[col_v.at[g]],
                              sem_d).wait()
        return 0
    lax.fori_loop(0, NB, deg_drain, 0)
    plsc.subcore_barrier()

    pltpu.sync_copy(deg_sh.at[pl.ds(s * NPT, NPT)],
                    deg_out.at[c, pl.ds(s * NPT, NPT)])


def _dis_col(dt):
    deg = dt[:, 0:1] + dt[:, 1:2]
    return jnp.where(deg > 0, lax.rsqrt(deg), 0.0)


def _tc_mm_dis_body(x_ref, w_ref, dt_ref, h_ref):
    dis = _dis_col(dt_ref[...])
    hp = jnp.dot(x_ref[...], w_ref[...],
                 preferred_element_type=jnp.float32) * dis
    h_ref[0] = hp
    h_ref[1] = hp


def _sc_agg_body(h_hbm, row_hbm, col_hbm, ew_hbm, out_hbm,
                 row_b0, row_b1, col_b0, col_b1, ew_b0, ew_b1,
                 cs_b0, cs_b1, cs_b2, ew_p,
                 rows_b0, rows_b1, rows_b2, acc_sh,
                 sem_i0, sem_i1, sem_r0, sem_r1, sem_r2,
                 sem_s0, sem_s1, sem_s2):
    c = lax.axis_index("c")
    s = lax.axis_index("s")

    rbuf = (row_b0, row_b1)
    cbuf = (col_b0, col_b1)
    ebuf = (ew_b0, ew_b1)
    csb = (cs_b0, cs_b1, cs_b2)
    gbuf = (rows_b0, rows_b1, rows_b2)
    sem_i = (sem_i0, sem_i1)
    sem_r = (sem_r0, sem_r1, sem_r2)
    sem_s = (sem_s0, sem_s1, sem_s2)

    def zero_rows(r, _):
        for j in range(8):
            rows_b0[r, pl.ds(j * LN, LN)] = jnp.zeros((LN,), jnp.float32)
        return 0
    lax.fori_loop(0, B, zero_rows, 0)
    for off in range(0, NPT - B + 1, B):
        pltpu.sync_copy(rows_b0, acc_sh.at[pl.ds(s * NPT + off, B)])
    rem = NPT % B
    if rem:
        pltpu.sync_copy(rows_b0.at[pl.ds(0, rem)],
                        acc_sh.at[pl.ds(s * NPT + NPT - rem, rem)])
    plsc.subcore_barrier()

    def issue_idx(g, par):
        pltpu.async_copy(row_hbm.at[s, c, g], rbuf[par], sem_i[par])
        pltpu.async_copy(col_hbm.at[s, c, g], cbuf[par], sem_i[par])
        pltpu.async_copy(ew_hbm.at[s, c, g], ebuf[par], sem_i[par])

    def wait_idx(g, par):
        pltpu.make_async_copy(row_hbm.at[s, c, g], rbuf[par],
                              sem_i[par]).wait()
        pltpu.make_async_copy(col_hbm.at[s, c, g], cbuf[par],
                              sem_i[par]).wait()
        pltpu.make_async_copy(ew_hbm.at[s, c, g], ebuf[par],
                              sem_i[par]).wait()

    def wait_scatter(p3):
        pltpu.make_async_copy(gbuf[p3], acc_sh.at[csb[p3]],
                              sem_s[p3]).wait()

    issue_idx(0, 0)
    issue_idx(1, 1)
    wait_idx(0, 0)
    pltpu.async_copy(h_hbm.at[c].at[rbuf[0]], gbuf[0], sem_r[0])

    def step(i, _):
        for t in range(6):
            g = 6 * i + t
            p2, p3, q2, q3 = t % 2, t % 3, (t + 1) % 2, (t + 1) % 3
            for k in range(B // LN):
                csb[p3][pl.ds(k * LN, LN)] = cbuf[p2][pl.ds(k * LN, LN)]
                ew_p[pl.ds(k * LN, LN)] = ebuf[p2][pl.ds(k * LN, LN)]
            pltpu.make_async_copy(h_hbm.at[c].at[rbuf[p2]], gbuf[p3],
                                  sem_r[p3]).wait()
            @pl.when(g + 2 < NB)
            def _():
                issue_idx(g + 2, p2)
            @pl.when(g + 1 < NB)
            def _():
                wait_idx(g + 1, q2)

            if t >= 2:
                wait_scatter(q3)
            else:
                @pl.when(g >= 2)
                def _():
                    wait_scatter(q3)

            @pl.when(g + 1 < NB)
            def _():
                pltpu.async_copy(h_hbm.at[c].at[rbuf[q2]], gbuf[q3], sem_r[q3])

            def scale_edge(b, _):
                m = ew_p[pl.ds(b, LN)][0]
                for j in range(8):
                    gbuf[p3][b, pl.ds(j * LN, LN)] = (
                        gbuf[p3][b, pl.ds(j * LN, LN)] * m)
                return 0
            lax.fori_loop(0, B, scale_edge, 0, unroll=16)

            pltpu.async_copy(gbuf[p3], acc_sh.at[csb[p3]], sem_s[p3],
                             add=True)
        return 0
    lax.fori_loop(0, NB // 6, step, 0)

    wait_scatter((NB - 2) % 3)
    wait_scatter((NB - 1) % 3)
    plsc.subcore_barrier()

    pltpu.sync_copy(acc_sh.at[pl.ds(s * NPT, NPT)],
                    out_hbm.at[c, pl.ds(s * NPT, NPT)])


def _tc_gates_body(p_ref, dt_ref, bx_ref, bh_ref, bi_ref, bc_ref, bo_ref,
                   o_ref):
    dis = _dis_col(dt_ref[...])
    s = (p_ref[0] + p_ref[1]) * dis + bx_ref[...] + bh_ref[...]
    gi = jax.nn.sigmoid(s + bi_ref[...])
    gt = jnp.tanh(s + bc_ref[...])
    go = jax.nn.sigmoid(s + bo_ref[...])
    o_ref[...] = go * jnp.tanh(gi * gt)


def kernel(X, edge_index, edge_weight, W_x, b_x, W_h, b_h, b_i, b_f, b_c, b_o):
    n, d = X.shape
    e = edge_weight.shape[0]

    pad = EPAD - e
    row = jnp.concatenate([edge_index[0], jnp.zeros((pad,), jnp.int32)])
    col = jnp.concatenate([edge_index[1], jnp.zeros((pad,), jnp.int32)])
    ew = jnp.concatenate([edge_weight, jnp.zeros((pad,), jnp.float32)])
    row4 = row.reshape(NS, NC, NB, B)
    col4 = col.reshape(NS, NC, NB, B)
    ew4 = ew.reshape(NS, NC, NB, B)

    mesh = plsc.VectorSubcoreMesh(core_axis_name="c", subcore_axis_name="s")

    sc_deg = pl.kernel(
        _sc_deg_body,
        out_type=jax.ShapeDtypeStruct((NC, NPAD), jnp.float32),
        mesh=mesh,
        scratch_types=[
            pltpu.VMEM((NB, B), jnp.int32),
            pltpu.VMEM((NB, B), jnp.float32),
            pltpu.VMEM((NPT,), jnp.float32),
            pltpu.VMEM_SHARED((NPAD,), jnp.float32),
            pltpu.SemaphoreType.DMA,
        ],
    )
    deg_part = sc_deg(col4, ew4)
    deg_t = deg_part.T

    bn = 1024
    grid = NPAD // bn
    hp = pl.pallas_call(
        _tc_mm_dis_body,
        grid=(grid,),
        in_specs=[pl.BlockSpec((bn, d), lambda i: (i, 0)),
                  pl.BlockSpec((d, d), lambda i: (0, 0)),
                  pl.BlockSpec((bn, NC), lambda i: (i, 0))],
        out_specs=pl.BlockSpec((NC, bn, d), lambda i: (0, i, 0)),
        out_shape=jax.ShapeDtypeStruct((NC, n, d), jnp.float32),
    )(X, W_x, deg_t)

    sc_agg = pl.kernel(
        _sc_agg_body,
        out_type=jax.ShapeDtypeStruct((NC, NPAD, d), jnp.float32),
        mesh=mesh,
        scratch_types=[
            pltpu.VMEM((B,), jnp.int32),
            pltpu.VMEM((B,), jnp.int32),
            pltpu.VMEM((B,), jnp.int32),
            pltpu.VMEM((B,), jnp.int32),
            pltpu.VMEM((B,), jnp.float32),
            pltpu.VMEM((B,), jnp.float32),
            pltpu.VMEM((B,), jnp.int32),
            pltpu.VMEM((B,), jnp.int32),
            pltpu.VMEM((B,), jnp.int32),
            pltpu.VMEM((B + LN,), jnp.float32),
            pltpu.VMEM((B, d), jnp.float32),
            pltpu.VMEM((B, d), jnp.float32),
            pltpu.VMEM((B, d), jnp.float32),
            pltpu.VMEM_SHARED((NPAD, d), jnp.float32),
            pltpu.SemaphoreType.DMA,
            pltpu.SemaphoreType.DMA,
            pltpu.SemaphoreType.DMA,
            pltpu.SemaphoreType.DMA,
            pltpu.SemaphoreType.DMA,
            pltpu.SemaphoreType.DMA,
            pltpu.SemaphoreType.DMA,
            pltpu.SemaphoreType.DMA,
        ],
    )
    part = sc_agg(hp, row4, col4, ew4)

    bx = b_x.reshape(1, d).astype(jnp.float32)
    bh = b_h.reshape(1, d).astype(jnp.float32)
    bi = b_i.reshape(1, d).astype(jnp.float32)
    bc = b_c.reshape(1, d).astype(jnp.float32)
    bo = b_o.reshape(1, d).astype(jnp.float32)
    H = pl.pallas_call(
        _tc_gates_body,
        grid=(grid,),
        in_specs=[pl.BlockSpec((NC, bn, d), lambda i: (0, i, 0)),
                  pl.BlockSpec((bn, NC), lambda i: (i, 0)),
                  pl.BlockSpec((1, d), lambda i: (0, 0)),
                  pl.BlockSpec((1, d), lambda i: (0, 0)),
                  pl.BlockSpec((1, d), lambda i: (0, 0)),
                  pl.BlockSpec((1, d), lambda i: (0, 0)),
                  pl.BlockSpec((1, d), lambda i: (0, 0))],
        out_specs=pl.BlockSpec((bn, d), lambda i: (i, 0)),
        out_shape=jax.ShapeDtypeStruct((n, d), jnp.float32),
    )(part, deg_t, bx, bh, bi, bc, bo)
    return H


# speedup vs baseline: 1.1093x; 1.1093x over previous
import jax
import jax.numpy as jnp
from jax import lax
from jax.experimental import pallas as pl
from jax.experimental.pallas import tpu as pltpu
from jax.experimental.pallas import tpu_sc as plsc

NC = 2
NS = 16
LN = 16

B = 112
NB = 90
EPT = NB * B
EPAD = NC * NS * EPT

NPAD = 10240
NPT = NPAD // NS


def _sc_deg_body(col_hbm, ew_hbm, deg_out, col_v, ew_v, zvec, deg_sh,
                 sem_d):
    c = lax.axis_index("c")
    s = lax.axis_index("s")

    pltpu.sync_copy(col_hbm.at[s, c], col_v)
    pltpu.sync_copy(ew_hbm.at[s, c], ew_v)

    def zero_zvec(i, _):
        zvec[pl.ds(i * LN, LN)] = jnp.zeros((LN,), jnp.float32)
        return 0
    lax.fori_loop(0, NPT // LN, zero_zvec, 0)
    pltpu.sync_copy(zvec, deg_sh.at[pl.ds(s * NPT, NPT)])
    plsc.subcore_barrier()

    def deg_batch(g, _):
        pltpu.async_copy(ew_v.at[g], deg_sh.at[col_v.at[g]], sem_d, add=True)
        return 0
    lax.fori_loop(0, NB, deg_batch, 0)
    def deg_drain(g, _):
        pltpu.make_async_copy(ew_v.at[g], deg_sh.at[col_v.at[g]],
                              sem_d).wait()
        return 0
    lax.fori_loop(0, NB, deg_drain, 0)
    plsc.subcore_barrier()

    pltpu.sync_copy(deg_sh.at[pl.ds(s * NPT, NPT)],
                    deg_out.at[c, pl.ds(s * NPT, NPT)])


def _dis_col(dt):
    deg = dt[:, 0:1] + dt[:, 1:2]
    return jnp.where(deg > 0, lax.rsqrt(deg), 0.0)


def _tc_mm_dis_body(x_ref, w_ref, dt_ref, h_ref):
    dis = _dis_col(dt_ref[...])
    h_ref[...] = jnp.dot(x_ref[...], w_ref[...],
                         preferred_element_type=jnp.float32) * dis


def _sc_agg_body(h_hbm, row_hbm, col_hbm, ew_hbm, out_hbm,
                 row_b0, row_b1, col_b0, col_b1, ew_b0, ew_b1,
                 cs_b0, cs_b1, cs_b2, ew_p,
                 rows_b0, rows_b1, rows_b2, acc_sh,
                 sem_i0, sem_i1, sem_r0, sem_r1, sem_r2,
                 sem_s0, sem_s1, sem_s2):
    c = lax.axis_index("c")
    s = lax.axis_index("s")

    rbuf = (row_b0, row_b1)
    cbuf = (col_b0, col_b1)
    ebuf = (ew_b0, ew_b1)
    csb = (cs_b0, cs_b1, cs_b2)
    gbuf = (rows_b0, rows_b1, rows_b2)
    sem_i = (sem_i0, sem_i1)
    sem_r = (sem_r0, sem_r1, sem_r2)
    sem_s = (sem_s0, sem_s1, sem_s2)

    def zero_rows(r, _):
        for j in range(8):
            rows_b0[r, pl.ds(j * LN, LN)] = jnp.zeros((LN,), jnp.float32)
        return 0
    lax.fori_loop(0, B, zero_rows, 0)
    for off in range(0, NPT - B + 1, B):
        pltpu.sync_copy(rows_b0, acc_sh.at[pl.ds(s * NPT + off, B)])
    rem = NPT % B
    if rem:
        pltpu.sync_copy(rows_b0.at[pl.ds(0, rem)],
                        acc_sh.at[pl.ds(s * NPT + NPT - rem, rem)])
    plsc.subcore_barrier()

    def issue_idx(g, par):
        pltpu.async_copy(row_hbm.at[s, c, g], rbuf[par], sem_i[par])
        pltpu.async_copy(col_hbm.at[s, c, g], cbuf[par], sem_i[par])
        pltpu.async_copy(ew_hbm.at[s, c, g], ebuf[par], sem_i[par])

    def wait_idx(g, par):
        pltpu.make_async_copy(row_hbm.at[s, c, g], rbuf[par],
                              sem_i[par]).wait()
        pltpu.make_async_copy(col_hbm.at[s, c, g], cbuf[par],
                              sem_i[par]).wait()
        pltpu.make_async_copy(ew_hbm.at[s, c, g], ebuf[par],
                              sem_i[par]).wait()

    def wait_scatter(p3):
        pltpu.make_async_copy(gbuf[p3], acc_sh.at[csb[p3]],
                              sem_s[p3]).wait()

    issue_idx(0, 0)
    issue_idx(1, 1)
    wait_idx(0, 0)
    pltpu.async_copy(h_hbm.at[rbuf[0]], gbuf[0], sem_r[0])

    def step(i, _):
        for t in range(6):
            g = 6 * i + t
            p2, p3, q2, q3 = t % 2, t % 3, (t + 1) % 2, (t + 1) % 3
            for k in range(B // LN):
                csb[p3][pl.ds(k * LN, LN)] = cbuf[p2][pl.ds(k * LN, LN)]
                ew_p[pl.ds(k * LN, LN)] = ebuf[p2][pl.ds(k * LN, LN)]
            pltpu.make_async_copy(h_hbm.at[rbuf[p2]], gbuf[p3],
                                  sem_r[p3]).wait()
            @pl.when(g + 2 < NB)
            def _():
                issue_idx(g + 2, p2)
            @pl.when(g + 1 < NB)
            def _():
                wait_idx(g + 1, q2)

            if t >= 2:
                wait_scatter(q3)
            else:
                @pl.when(g >= 2)
                def _():
                    wait_scatter(q3)

            @pl.when(g + 1 < NB)
            def _():
                pltpu.async_copy(h_hbm.at[rbuf[q2]], gbuf[q3], sem_r[q3])

            def scale_edge(b, _):
                m = ew_p[pl.ds(b, LN)][0]
                for j in range(8):
                    gbuf[p3][b, pl.ds(j * LN, LN)] = (
                        gbuf[p3][b, pl.ds(j * LN, LN)] * m)
                return 0
            lax.fori_loop(0, B, scale_edge, 0, unroll=16)

            pltpu.async_copy(gbuf[p3], acc_sh.at[csb[p3]], sem_s[p3],
                             add=True)
        return 0
    lax.fori_loop(0, NB // 6, step, 0)

    wait_scatter((NB - 2) % 3)
    wait_scatter((NB - 1) % 3)
    plsc.subcore_barrier()

    pltpu.sync_copy(acc_sh.at[pl.ds(s * NPT, NPT)],
                    out_hbm.at[c, pl.ds(s * NPT, NPT)])


def _tc_gates_body(p_ref, dt_ref, bx_ref, bh_ref, bi_ref, bc_ref, bo_ref,
                   o_ref):
    dis = _dis_col(dt_ref[...])
    s = (p_ref[0] + p_ref[1]) * dis + bx_ref[...] + bh_ref[...]
    gi = jax.nn.sigmoid(s + bi_ref[...])
    gt = jnp.tanh(s + bc_ref[...])
    go = jax.nn.sigmoid(s + bo_ref[...])
    o_ref[...] = go * jnp.tanh(gi * gt)


def kernel(X, edge_index, edge_weight, W_x, b_x, W_h, b_h, b_i, b_f, b_c, b_o):
    n, d = X.shape
    e = edge_weight.shape[0]

    pad = EPAD - e
    row = jnp.concatenate([edge_index[0], jnp.zeros((pad,), jnp.int32)])
    col = jnp.concatenate([edge_index[1], jnp.zeros((pad,), jnp.int32)])
    ew = jnp.concatenate([edge_weight, jnp.zeros((pad,), jnp.float32)])
    row4 = row.reshape(NS, NC, NB, B)
    col4 = col.reshape(NS, NC, NB, B)
    ew4 = ew.reshape(NS, NC, NB, B)

    mesh = plsc.VectorSubcoreMesh(core_axis_name="c", subcore_axis_name="s")

    sc_deg = pl.kernel(
        _sc_deg_body,
        out_type=jax.ShapeDtypeStruct((NC, NPAD), jnp.float32),
        mesh=mesh,
        scratch_types=[
            pltpu.VMEM((NB, B), jnp.int32),
            pltpu.VMEM((NB, B), jnp.float32),
            pltpu.VMEM((NPT,), jnp.float32),
            pltpu.VMEM_SHARED((NPAD,), jnp.float32),
            pltpu.SemaphoreType.DMA,
        ],
    )
    deg_part = sc_deg(col4, ew4)
    deg_t = deg_part.T

    bn = 1024
    grid = NPAD // bn
    hp = pl.pallas_call(
        _tc_mm_dis_body,
        grid=(grid,),
        in_specs=[pl.BlockSpec((bn, d), lambda i: (i, 0)),
                  pl.BlockSpec((d, d), lambda i: (0, 0)),
                  pl.BlockSpec((bn, NC), lambda i: (i, 0))],
        out_specs=pl.BlockSpec((bn, d), lambda i: (i, 0)),
        out_shape=jax.ShapeDtypeStruct((n, d), jnp.float32),
    )(X, W_x, deg_t)

    sc_agg = pl.kernel(
        _sc_agg_body,
        out_type=jax.ShapeDtypeStruct((NC, NPAD, d), jnp.float32),
        mesh=mesh,
        scratch_types=[
            pltpu.VMEM((B,), jnp.int32),
            pltpu.VMEM((B,), jnp.int32),
            pltpu.VMEM((B,), jnp.int32),
            pltpu.VMEM((B,), jnp.int32),
            pltpu.VMEM((B,), jnp.float32),
            pltpu.VMEM((B,), jnp.float32),
            pltpu.VMEM((B,), jnp.int32),
            pltpu.VMEM((B,), jnp.int32),
            pltpu.VMEM((B,), jnp.int32),
            pltpu.VMEM((B + LN,), jnp.float32),
            pltpu.VMEM((B, d), jnp.float32),
            pltpu.VMEM((B, d), jnp.float32),
            pltpu.VMEM((B, d), jnp.float32),
            pltpu.VMEM_SHARED((NPAD, d), jnp.float32),
            pltpu.SemaphoreType.DMA,
            pltpu.SemaphoreType.DMA,
            pltpu.SemaphoreType.DMA,
            pltpu.SemaphoreType.DMA,
            pltpu.SemaphoreType.DMA,
            pltpu.SemaphoreType.DMA,
            pltpu.SemaphoreType.DMA,
            pltpu.SemaphoreType.DMA,
        ],
    )
    part = sc_agg(hp, row4, col4, ew4)

    bx = b_x.reshape(1, d).astype(jnp.float32)
    bh = b_h.reshape(1, d).astype(jnp.float32)
    bi = b_i.reshape(1, d).astype(jnp.float32)
    bc = b_c.reshape(1, d).astype(jnp.float32)
    bo = b_o.reshape(1, d).astype(jnp.float32)
    H = pl.pallas_call(
        _tc_gates_body,
        grid=(grid,),
        in_specs=[pl.BlockSpec((NC, bn, d), lambda i: (0, i, 0)),
                  pl.BlockSpec((bn, NC), lambda i: (i, 0)),
                  pl.BlockSpec((1, d), lambda i: (0, 0)),
                  pl.BlockSpec((1, d), lambda i: (0, 0)),
                  pl.BlockSpec((1, d), lambda i: (0, 0)),
                  pl.BlockSpec((1, d), lambda i: (0, 0)),
                  pl.BlockSpec((1, d), lambda i: (0, 0))],
        out_specs=pl.BlockSpec((bn, d), lambda i: (i, 0)),
        out_shape=jax.ShapeDtypeStruct((n, d), jnp.float32),
    )(part, deg_t, bx, bh, bi, bc, bo)
    return H


# speedup vs baseline: 1.2041x; 1.0854x over previous
import jax
import jax.numpy as jnp
from jax import lax
from jax.experimental import pallas as pl
from jax.experimental.pallas import tpu as pltpu
from jax.experimental.pallas import tpu_sc as plsc

NC = 2
NS = 16
LN = 16

B = 112
NB = 90
EPT = NB * B
EPAD = NC * NS * EPT

NPAD = 10240
NPT = NPAD // NS

NB0 = 108
NB1 = 2 * NB - NB0


def _sc_deg_body(col_hbm, ew_hbm, deg_out, col_v, ew_v, zvec, deg_sh,
                 sem_d):
    c = lax.axis_index("c")
    s = lax.axis_index("s")

    pltpu.sync_copy(col_hbm.at[s, c], col_v)
    pltpu.sync_copy(ew_hbm.at[s, c], ew_v)

    def zero_zvec(i, _):
        zvec[pl.ds(i * LN, LN)] = jnp.zeros((LN,), jnp.float32)
        return 0
    lax.fori_loop(0, NPT // LN, zero_zvec, 0)
    pltpu.sync_copy(zvec, deg_sh.at[pl.ds(s * NPT, NPT)])
    plsc.subcore_barrier()

    def deg_batch(g, _):
        pltpu.async_copy(ew_v.at[g], deg_sh.at[col_v.at[g]], sem_d, add=True)
        return 0
    lax.fori_loop(0, NB, deg_batch, 0)
    def deg_drain(g, _):
        pltpu.make_async_copy(ew_v.at[g], deg_sh.at[col_v.at[g]],
                              sem_d).wait()
        return 0
    lax.fori_loop(0, NB, deg_drain, 0)
    plsc.subcore_barrier()

    pltpu.sync_copy(deg_sh.at[pl.ds(s * NPT, NPT)],
                    deg_out.at[c, pl.ds(s * NPT, NPT)])


def _dis_col(dt):
    deg = dt[:, 0:1] + dt[:, 1:2]
    return jnp.where(deg > 0, lax.rsqrt(deg), 0.0)


def _tc_mm_dis_body(x_ref, w_ref, dt_ref, h_ref):
    dis = _dis_col(dt_ref[...])
    h_ref[...] = jnp.dot(x_ref[...], w_ref[...],
                         preferred_element_type=jnp.float32) * dis


def _sc_agg_body(h_hbm, row_hbm, col_hbm, ew_hbm, out_hbm,
                 row_b0, row_b1, col_b0, col_b1, ew_b0, ew_b1,
                 cs_b0, cs_b1, cs_b2, ew_p,
                 rows_b0, rows_b1, rows_b2, acc_sh,
                 sem_i0, sem_i1, sem_r0, sem_r1, sem_r2,
                 sem_s0, sem_s1, sem_s2):
    c = lax.axis_index("c")
    s = lax.axis_index("s")
    nb = jnp.where(c == 0, NB0, NB1)
    base_b = jnp.where(c == 0, s * NB0, NS * NB0 + s * NB1)

    rbuf = (row_b0, row_b1)
    cbuf = (col_b0, col_b1)
    ebuf = (ew_b0, ew_b1)
    csb = (cs_b0, cs_b1, cs_b2)
    gbuf = (rows_b0, rows_b1, rows_b2)
    sem_i = (sem_i0, sem_i1)
    sem_r = (sem_r0, sem_r1, sem_r2)
    sem_s = (sem_s0, sem_s1, sem_s2)

    def zero_rows(r, _):
        for j in range(8):
            rows_b0[r, pl.ds(j * LN, LN)] = jnp.zeros((LN,), jnp.float32)
        return 0
    lax.fori_loop(0, B, zero_rows, 0)
    for off in range(0, NPT - B + 1, B):
        pltpu.sync_copy(rows_b0, acc_sh.at[pl.ds(s * NPT + off, B)])
    rem = NPT % B
    if rem:
        pltpu.sync_copy(rows_b0.at[pl.ds(0, rem)],
                        acc_sh.at[pl.ds(s * NPT + NPT - rem, rem)])
    plsc.subcore_barrier()

    def issue_idx(g, par):
        o = (base_b + g) * B
        pltpu.async_copy(row_hbm.at[pl.ds(o, B)], rbuf[par], sem_i[par])
        pltpu.async_copy(col_hbm.at[pl.ds(o, B)], cbuf[par], sem_i[par])
        pltpu.async_copy(ew_hbm.at[pl.ds(o, B)], ebuf[par], sem_i[par])

    def wait_idx(g, par):
        o = (base_b + g) * B
        pltpu.make_async_copy(row_hbm.at[pl.ds(o, B)], rbuf[par],
                              sem_i[par]).wait()
        pltpu.make_async_copy(col_hbm.at[pl.ds(o, B)], cbuf[par],
                              sem_i[par]).wait()
        pltpu.make_async_copy(ew_hbm.at[pl.ds(o, B)], ebuf[par],
                              sem_i[par]).wait()

    def wait_scatter(p3):
        pltpu.make_async_copy(gbuf[p3], acc_sh.at[csb[p3]],
                              sem_s[p3]).wait()

    issue_idx(0, 0)
    issue_idx(1, 1)
    wait_idx(0, 0)
    pltpu.async_copy(h_hbm.at[rbuf[0]], gbuf[0], sem_r[0])

    def step(i, _):
        for t in range(6):
            g = 6 * i + t
            p2, p3, q2, q3 = t % 2, t % 3, (t + 1) % 2, (t + 1) % 3
            for k in range(B // LN):
                csb[p3][pl.ds(k * LN, LN)] = cbuf[p2][pl.ds(k * LN, LN)]
                ew_p[pl.ds(k * LN, LN)] = ebuf[p2][pl.ds(k * LN, LN)]
            pltpu.make_async_copy(h_hbm.at[rbuf[p2]], gbuf[p3],
                                  sem_r[p3]).wait()
            @pl.when(g + 2 < nb)
            def _():
                issue_idx(g + 2, p2)
            @pl.when(g + 1 < nb)
            def _():
                wait_idx(g + 1, q2)

            if t >= 2:
                wait_scatter(q3)
            else:
                @pl.when(g >= 2)
                def _():
                    wait_scatter(q3)

            @pl.when(g + 1 < nb)
            def _():
                pltpu.async_copy(h_hbm.at[rbuf[q2]], gbuf[q3], sem_r[q3])

            def scale_edge(b, _):
                m = ew_p[pl.ds(b, LN)][0]
                for j in range(8):
                    gbuf[p3][b, pl.ds(j * LN, LN)] = (
                        gbuf[p3][b, pl.ds(j * LN, LN)] * m)
                return 0
            lax.fori_loop(0, B, scale_edge, 0, unroll=16)

            pltpu.async_copy(gbuf[p3], acc_sh.at[csb[p3]], sem_s[p3],
                             add=True)
        return 0
    lax.fori_loop(0, nb // 6, step, 0)

    wait_scatter(1)
    wait_scatter(2)
    plsc.subcore_barrier()

    pltpu.sync_copy(acc_sh.at[pl.ds(s * NPT, NPT)],
                    out_hbm.at[c, pl.ds(s * NPT, NPT)])


def _tc_gates_body(p_ref, dt_ref, bx_ref, bh_ref, bi_ref, bc_ref, bo_ref,
                   o_ref):
    dis = _dis_col(dt_ref[...])
    s = (p_ref[0] + p_ref[1]) * dis + bx_ref[...] + bh_ref[...]
    gi = jax.nn.sigmoid(s + bi_ref[...])
    gt = jnp.tanh(s + bc_ref[...])
    go = jax.nn.sigmoid(s + bo_ref[...])
    o_ref[...] = go * jnp.tanh(gi * gt)


def kernel(X, edge_index, edge_weight, W_x, b_x, W_h, b_h, b_i, b_f, b_c, b_o):
    n, d = X.shape
    e = edge_weight.shape[0]

    pad = EPAD - e
    row = jnp.concatenate([edge_index[0], jnp.zeros((pad,), jnp.int32)])
    col = jnp.concatenate([edge_index[1], jnp.zeros((pad,), jnp.int32)])
    ew = jnp.concatenate([edge_weight, jnp.zeros((pad,), jnp.float32)])
    row4 = row.reshape(NS, NC, NB, B)
    col4 = col.reshape(NS, NC, NB, B)
    ew4 = ew.reshape(NS, NC, NB, B)

    mesh = plsc.VectorSubcoreMesh(core_axis_name="c", subcore_axis_name="s")

    sc_deg = pl.kernel(
        _sc_deg_body,
        out_type=jax.ShapeDtypeStruct((NC, NPAD), jnp.float32),
        mesh=mesh,
        scratch_types=[
            pltpu.VMEM((NB, B), jnp.int32),
            pltpu.VMEM((NB, B), jnp.float32),
            pltpu.VMEM((NPT,), jnp.float32),
            pltpu.VMEM_SHARED((NPAD,), jnp.float32),
            pltpu.SemaphoreType.DMA,
        ],
    )
    deg_part = sc_deg(col4, ew4)
    deg_t = deg_part.T

    bn = 1024
    grid = NPAD // bn
    hp = pl.pallas_call(
        _tc_mm_dis_body,
        grid=(grid,),
        in_specs=[pl.BlockSpec((bn, d), lambda i: (i, 0)),
                  pl.BlockSpec((d, d), lambda i: (0, 0)),
                  pl.BlockSpec((bn, NC), lambda i: (i, 0))],
        out_specs=pl.BlockSpec((bn, d), lambda i: (i, 0)),
        out_shape=jax.ShapeDtypeStruct((n, d), jnp.float32),
    )(X, W_x, deg_t)

    sc_agg = pl.kernel(
        _sc_agg_body,
        out_type=jax.ShapeDtypeStruct((NC, NPAD, d), jnp.float32),
        mesh=mesh,
        scratch_types=[
            pltpu.VMEM((B,), jnp.int32),
            pltpu.VMEM((B,), jnp.int32),
            pltpu.VMEM((B,), jnp.int32),
            pltpu.VMEM((B,), jnp.int32),
            pltpu.VMEM((B,), jnp.float32),
            pltpu.VMEM((B,), jnp.float32),
            pltpu.VMEM((B,), jnp.int32),
            pltpu.VMEM((B,), jnp.int32),
            pltpu.VMEM((B,), jnp.int32),
            pltpu.VMEM((B + LN,), jnp.float32),
            pltpu.VMEM((B, d), jnp.float32),
            pltpu.VMEM((B, d), jnp.float32),
            pltpu.VMEM((B, d), jnp.float32),
            pltpu.VMEM_SHARED((NPAD, d), jnp.float32),
            pltpu.SemaphoreType.DMA,
            pltpu.SemaphoreType.DMA,
            pltpu.SemaphoreType.DMA,
            pltpu.SemaphoreType.DMA,
            pltpu.SemaphoreType.DMA,
            pltpu.SemaphoreType.DMA,
            pltpu.SemaphoreType.DMA,
            pltpu.SemaphoreType.DMA,
        ],
    )
    part = sc_agg(hp, row, col, ew)

    bx = b_x.reshape(1, d).astype(jnp.float32)
    bh = b_h.reshape(1, d).astype(jnp.float32)
    bi = b_i.reshape(1, d).astype(jnp.float32)
    bc = b_c.reshape(1, d).astype(jnp.float32)
    bo = b_o.reshape(1, d).astype(jnp.float32)
    H = pl.pallas_call(
        _tc_gates_body,
        grid=(grid,),
        in_specs=[pl.BlockSpec((NC, bn, d), lambda i: (0, i, 0)),
                  pl.BlockSpec((bn, NC), lambda i: (i, 0)),
                  pl.BlockSpec((1, d), lambda i: (0, 0)),
                  pl.BlockSpec((1, d), lambda i: (0, 0)),
                  pl.BlockSpec((1, d), lambda i: (0, 0)),
                  pl.BlockSpec((1, d), lambda i: (0, 0)),
                  pl.BlockSpec((1, d), lambda i: (0, 0))],
        out_specs=pl.BlockSpec((bn, d), lambda i: (i, 0)),
        out_shape=jax.ShapeDtypeStruct((n, d), jnp.float32),
    )(part, deg_t, bx, bh, bi, bc, bo)
    return H
